# baseline (speedup 1.0000x reference)
import jax
import jax.numpy as jnp
from jax import lax
from jax.experimental import pallas as pl
from jax.experimental.pallas import tpu as pltpu
from jax.experimental.pallas import tpu_sc as plsc

V = 1000000
D = 16
H = 64
B = 4096
L = 50

NC, NS = 2, 16
NW = NC * NS
ROWS = B * L
RPW = ROWS // NW
CHUNK = 128
NCHUNK = RPW // CHUNK


def _gather_body(emb_hbm, idx_hbm, out_hbm, idx_v, rows_v, sem):
    wid = lax.axis_index("s") * NC + lax.axis_index("c")
    pltpu.sync_copy(idx_hbm.at[wid], idx_v)

    def fire(j, carry):
        pltpu.async_copy(emb_hbm.at[idx_v.at[j]], rows_v.at[j], sem)
        return carry

    lax.fori_loop(0, NCHUNK, fire, 0)

    def drain(j, carry):
        pltpu.make_async_copy(emb_hbm.at[idx_v.at[j]], rows_v.at[j],
                              sem).wait()
        return carry

    lax.fori_loop(0, NCHUNK, drain, 0)
    pltpu.sync_copy(rows_v, out_hbm.at[pl.ds(wid * NCHUNK, NCHUNK)])


_gather_cache = []


def _gather(emb, idx):
    if not _gather_cache:
        _gather_cache.append(pl.kernel(
            _gather_body,
            out_type=jax.ShapeDtypeStruct((NW * NCHUNK, CHUNK, D),
                                          emb.dtype),
            mesh=plsc.VectorSubcoreMesh(
                core_axis_name="c", subcore_axis_name="s",
                num_cores=NC, num_subcores=NS),
            scratch_types=[
                pltpu.VMEM((NCHUNK, CHUNK), jnp.int32),
                pltpu.VMEM((NCHUNK, CHUNK, D), emb.dtype),
                pltpu.SemaphoreType.DMA,
            ],
            compiler_params=pltpu.CompilerParams(use_tc_tiling_on_sc=False),
        ))
    return _gather_cache[0](emb, idx)


BB = 2048


def _lstm_mlp_body(e_ref, wih_ref, whh_ref, bg_ref, w1_ref, b1_ref,
                   w2_ref, b2_ref, w3_ref, b3_ref, w4_ref, b4_ref,
                   w5_ref, b5_ref, out_ref, et_ref):
    et_ref[:] = jnp.transpose(e_ref[:], (1, 0))

    def step(t, hc):
        h, c = hc
        xt = et_ref[pl.ds(t * D, D), :]
        g = (jnp.dot(wih_ref[:], xt, preferred_element_type=jnp.float32)
             + jnp.dot(whh_ref[:], h, preferred_element_type=jnp.float32)
             + bg_ref[:])
        i_g = jax.nn.sigmoid(g[0:H])
        f_g = jax.nn.sigmoid(g[H:2 * H])
        g_g = jnp.tanh(g[2 * H:3 * H])
        o_g = jax.nn.sigmoid(g[3 * H:4 * H])
        c = f_g * c + i_g * g_g
        h = o_g * jnp.tanh(c)
        return (h, c)

    h0 = jnp.zeros((H, BB), jnp.float32)
    c0 = jnp.zeros((H, BB), jnp.float32)
    h, _ = lax.fori_loop(0, L, step, (h0, c0))

    a = jax.nn.relu(h)
    a = jax.nn.relu(jnp.dot(w1_ref[:], a, preferred_element_type=jnp.float32)
                    + b1_ref[:])
    a = jax.nn.relu(jnp.dot(w2_ref[:], a, preferred_element_type=jnp.float32)
                    + b2_ref[:])
    a = jax.nn.relu(jnp.dot(w3_ref[:], a, preferred_element_type=jnp.float32)
                    + b3_ref[:])
    a = jax.nn.relu(jnp.dot(w4_ref[:], a, preferred_element_type=jnp.float32)
                    + b4_ref[:])
    a = jax.nn.sigmoid(jnp.dot(w5_ref[:], a, preferred_element_type=jnp.float32)
                       + b5_ref[:])
    out_ref[:] = a


def _full(shape):
    return pl.BlockSpec(shape, lambda *_: tuple(0 for _ in shape))


def _lstm_mlp(e, wih, whh, bg, w1, b1, w2, b2, w3, b3, w4, b4, w5, b5,
              interpret=False):
    return pl.pallas_call(
        _lstm_mlp_body,
        grid=(B // BB,),
        scratch_shapes=[pltpu.VMEM((L * D, BB), jnp.float32)],
        in_specs=[
            pl.BlockSpec((BB, L * D), lambda i: (i, 0)),
            _full(wih.shape), _full(whh.shape), _full(bg.shape),
            _full(w1.shape), _full(b1.shape),
            _full(w2.shape), _full(b2.shape),
            _full(w3.shape), _full(b3.shape),
            _full(w4.shape), _full(b4.shape),
            _full(w5.shape), _full(b5.shape),
        ],
        out_specs=pl.BlockSpec((1, BB), lambda i: (0, i)),
        out_shape=jax.ShapeDtypeStruct((1, B), jnp.float32),
        interpret=interpret,
    )(e, wih, whh, bg, w1, b1, w2, b2, w3, b3, w4, b4, w5, b5)


def kernel(x, emb, W_ih, W_hh, b_ih, b_hh, W1, b1, W2, b2, W3, b3, W4, b4,
           W5, b5):
    idx = x.astype(jnp.int32).reshape(NW, NCHUNK, CHUNK)
    e = _gather(emb.astype(jnp.bfloat16), idx)
    e = e.reshape(B, L * D).astype(jnp.float32)
    bg = (b_ih + b_hh).reshape(4 * H, 1)
    out = _lstm_mlp(
        e, W_ih, W_hh, bg,
        W1, b1.reshape(-1, 1), W2, b2.reshape(-1, 1),
        W3, b3.reshape(-1, 1), W4, b4.reshape(-1, 1),
        W5, b5.reshape(1, 1))
    return out.reshape(B, 1)


# speedup vs baseline: 1.3670x; 1.3670x over previous
import jax
import jax.numpy as jnp
from jax import lax
from jax.experimental import pallas as pl
from jax.experimental.pallas import tpu as pltpu
from jax.experimental.pallas import tpu_sc as plsc

V = 1000000
D = 16
H = 64
B = 4096
L = 50

NC, NS = 2, 16
NW = NC * NS
ROWS = B * L
RPW = ROWS // NW
CHUNK = 128
NCHUNK = RPW // CHUNK


def _gather_body(emb_hbm, idx_hbm, out_hbm, idx_v, rows_v, sem):
    wid = lax.axis_index("s") * NC + lax.axis_index("c")
    pltpu.sync_copy(idx_hbm.at[wid], idx_v)

    def fire(j, carry):
        pltpu.async_copy(emb_hbm.at[idx_v.at[j]], rows_v.at[j], sem)
        return carry

    lax.fori_loop(0, NCHUNK, fire, 0)

    def drain(j, carry):
        pltpu.make_async_copy(emb_hbm.at[idx_v.at[j]], rows_v.at[j],
                              sem).wait()
        return carry

    lax.fori_loop(0, NCHUNK, drain, 0)
    pltpu.sync_copy(rows_v, out_hbm.at[pl.ds(wid * NCHUNK, NCHUNK)])


_gather_cache = []


def _gather(emb, idx):
    if not _gather_cache:
        _gather_cache.append(pl.kernel(
            _gather_body,
            out_type=jax.ShapeDtypeStruct((NW * NCHUNK, CHUNK, D),
                                          emb.dtype),
            mesh=plsc.VectorSubcoreMesh(
                core_axis_name="c", subcore_axis_name="s",
                num_cores=NC, num_subcores=NS),
            scratch_types=[
                pltpu.VMEM((NCHUNK, CHUNK), jnp.int32),
                pltpu.VMEM((NCHUNK, CHUNK, D), emb.dtype),
                pltpu.SemaphoreType.DMA,
            ],
            compiler_params=pltpu.CompilerParams(use_tc_tiling_on_sc=False),
        ))
    return _gather_cache[0](emb, idx)


BB = 2048


def _lstm_mlp_body(e_ref, wc_ref, bg_ref, w1_ref, b1_ref,
                   w2_ref, b2_ref, w3_ref, b3_ref, w4_ref, b4_ref,
                   w5_ref, b5_ref, out_ref, et_ref, xh_ref):
    et_ref[:] = jnp.transpose(e_ref[:], (1, 0))

    def step(t, hc):
        h, c = hc
        xh_ref[0:D, :] = et_ref[pl.ds(t * D, D), :]
        xh_ref[D:D + H, :] = h
        g = (jnp.dot(wc_ref[:], xh_ref[:],
                     preferred_element_type=jnp.float32)
             + bg_ref[:])
        i_g = jax.nn.sigmoid(g[0:H])
        f_g = jax.nn.sigmoid(g[H:2 * H])
        g_g = jnp.tanh(g[2 * H:3 * H])
        o_g = jax.nn.sigmoid(g[3 * H:4 * H])
        c = f_g * c + i_g * g_g
        h = o_g * jnp.tanh(c)
        return (h, c)

    h0 = jnp.zeros((H, BB), jnp.float32)
    c0 = jnp.zeros((H, BB), jnp.float32)
    h, _ = lax.fori_loop(0, L, step, (h0, c0))

    a = jax.nn.relu(h)
    a = jax.nn.relu(jnp.dot(w1_ref[:], a, preferred_element_type=jnp.float32)
                    + b1_ref[:])
    a = jax.nn.relu(jnp.dot(w2_ref[:], a, preferred_element_type=jnp.float32)
                    + b2_ref[:])
    a = jax.nn.relu(jnp.dot(w3_ref[:], a, preferred_element_type=jnp.float32)
                    + b3_ref[:])
    a = jax.nn.relu(jnp.dot(w4_ref[:], a, preferred_element_type=jnp.float32)
                    + b4_ref[:])
    a = jax.nn.sigmoid(jnp.dot(w5_ref[:], a, preferred_element_type=jnp.float32)
                       + b5_ref[:])
    out_ref[:] = a


def _full(shape):
    return pl.BlockSpec(shape, lambda *_: tuple(0 for _ in shape))


def _lstm_mlp(e, wc, bg, w1, b1, w2, b2, w3, b3, w4, b4, w5, b5,
              interpret=False):
    return pl.pallas_call(
        _lstm_mlp_body,
        grid=(B // BB,),
        scratch_shapes=[pltpu.VMEM((L * D, BB), jnp.float32),
                        pltpu.VMEM((D + H, BB), jnp.float32)],
        in_specs=[
            pl.BlockSpec((BB, L * D), lambda i: (i, 0)),
            _full(wc.shape), _full(bg.shape),
            _full(w1.shape), _full(b1.shape),
            _full(w2.shape), _full(b2.shape),
            _full(w3.shape), _full(b3.shape),
            _full(w4.shape), _full(b4.shape),
            _full(w5.shape), _full(b5.shape),
        ],
        out_specs=pl.BlockSpec((1, BB), lambda i: (0, i)),
        out_shape=jax.ShapeDtypeStruct((1, B), jnp.float32),
        interpret=interpret,
    )(e, wc, bg, w1, b1, w2, b2, w3, b3, w4, b4, w5, b5)


def kernel(x, emb, W_ih, W_hh, b_ih, b_hh, W1, b1, W2, b2, W3, b3, W4, b4,
           W5, b5):
    idx = x.astype(jnp.int32).reshape(NW, NCHUNK, CHUNK)
    e = _gather(emb, idx)
    e = e.reshape(B, L * D)
    bg = (b_ih + b_hh).reshape(4 * H, 1)
    wc = jnp.concatenate([W_ih, W_hh], axis=1)
    out = _lstm_mlp(
        e, wc, bg,
        W1, b1.reshape(-1, 1), W2, b2.reshape(-1, 1),
        W3, b3.reshape(-1, 1), W4, b4.reshape(-1, 1),
        W5, b5.reshape(1, 1))
    return out.reshape(B, 1)


# speedup vs baseline: 1.4072x; 1.0294x over previous
import jax
import jax.numpy as jnp
from jax import lax
from jax.experimental import pallas as pl
from jax.experimental.pallas import tpu as pltpu
from jax.experimental.pallas import tpu_sc as plsc

V = 1000000
D = 16
H = 64
B = 4096
L = 50

NC, NS = 2, 16
NW = NC * NS
ROWS = B * L
RPW = ROWS // NW
CHUNK = 128
NCHUNK = RPW // CHUNK


def _gather_body(emb_hbm, idx_hbm, out_hbm, idx_v, rows_v, sem):
    wid = lax.axis_index("s") * NC + lax.axis_index("c")
    pltpu.sync_copy(idx_hbm.at[wid], idx_v)

    def fire(j, carry):
        pltpu.async_copy(emb_hbm.at[idx_v.at[j]], rows_v.at[j], sem)
        return carry

    lax.fori_loop(0, NCHUNK, fire, 0)

    def drain(j, carry):
        pltpu.make_async_copy(emb_hbm.at[idx_v.at[j]], rows_v.at[j],
                              sem).wait()
        return carry

    lax.fori_loop(0, NCHUNK, drain, 0)
    pltpu.sync_copy(rows_v, out_hbm.at[pl.ds(wid * NCHUNK, NCHUNK)])


_gather_cache = []


def _gather(emb, idx):
    if not _gather_cache:
        _gather_cache.append(pl.kernel(
            _gather_body,
            out_type=jax.ShapeDtypeStruct((NW * NCHUNK, CHUNK, D),
                                          emb.dtype),
            mesh=plsc.VectorSubcoreMesh(
                core_axis_name="c", subcore_axis_name="s",
                num_cores=NC, num_subcores=NS),
            scratch_types=[
                pltpu.VMEM((NCHUNK, CHUNK), jnp.int32),
                pltpu.VMEM((NCHUNK, CHUNK, D), emb.dtype),
                pltpu.SemaphoreType.DMA,
            ],
            compiler_params=pltpu.CompilerParams(use_tc_tiling_on_sc=False),
        ))
    return _gather_cache[0](emb, idx)


BB = 2048


def _lstm_mlp_body(e_ref, wc_ref, w1_ref, b1_ref,
                   w2_ref, b2_ref, w3_ref, b3_ref, w4_ref, b4_ref,
                   w5_ref, b5_ref, out_ref, et_ref, xh_ref):
    et_ref[:] = jnp.transpose(e_ref[:], (1, 0))
    xh_ref[D + H:D + H + 1, :] = jnp.ones((1, BB), jnp.float32)

    def step(t, hc):
        h, c = hc
        xh_ref[0:D, :] = et_ref[pl.ds(t * D, D), :]
        xh_ref[D:D + H, :] = h
        g = jnp.dot(wc_ref[:], xh_ref[:],
                    preferred_element_type=jnp.float32)
        i_g = 0.5 + 0.5 * jnp.tanh(0.5 * g[0:H])
        f_g = 0.5 + 0.5 * jnp.tanh(0.5 * g[H:2 * H])
        g_g = jnp.tanh(g[2 * H:3 * H])
        o_g = 0.5 + 0.5 * jnp.tanh(0.5 * g[3 * H:4 * H])
        c = f_g * c + i_g * g_g
        h = o_g * jnp.tanh(c)
        return (h, c)

    h0 = jnp.zeros((H, BB), jnp.float32)
    c0 = jnp.zeros((H, BB), jnp.float32)
    h, _ = lax.fori_loop(0, L, step, (h0, c0))

    a = jax.nn.relu(h)
    a = jax.nn.relu(jnp.dot(w1_ref[:], a, preferred_element_type=jnp.float32)
                    + b1_ref[:])
    a = jax.nn.relu(jnp.dot(w2_ref[:], a, preferred_element_type=jnp.float32)
                    + b2_ref[:])
    a = jax.nn.relu(jnp.dot(w3_ref[:], a, preferred_element_type=jnp.float32)
                    + b3_ref[:])
    a = jax.nn.relu(jnp.dot(w4_ref[:], a, preferred_element_type=jnp.float32)
                    + b4_ref[:])
    a = jax.nn.sigmoid(jnp.dot(w5_ref[:], a, preferred_element_type=jnp.float32)
                       + b5_ref[:])
    out_ref[:] = a


def _full(shape):
    return pl.BlockSpec(shape, lambda *_: tuple(0 for _ in shape))


def _lstm_mlp(e, wc, w1, b1, w2, b2, w3, b3, w4, b4, w5, b5,
              interpret=False):
    return pl.pallas_call(
        _lstm_mlp_body,
        grid=(B // BB,),
        scratch_shapes=[pltpu.VMEM((L * D, BB), jnp.float32),
                        pltpu.VMEM((D + H + 1, BB), jnp.float32)],
        in_specs=[
            pl.BlockSpec((BB, L * D), lambda i: (i, 0)),
            _full(wc.shape),
            _full(w1.shape), _full(b1.shape),
            _full(w2.shape), _full(b2.shape),
            _full(w3.shape), _full(b3.shape),
            _full(w4.shape), _full(b4.shape),
            _full(w5.shape), _full(b5.shape),
        ],
        out_specs=pl.BlockSpec((1, BB), lambda i: (0, i)),
        out_shape=jax.ShapeDtypeStruct((1, B), jnp.float32),
        interpret=interpret,
    )(e, wc, w1, b1, w2, b2, w3, b3, w4, b4, w5, b5)


def kernel(x, emb, W_ih, W_hh, b_ih, b_hh, W1, b1, W2, b2, W3, b3, W4, b4,
           W5, b5):
    idx = x.astype(jnp.int32).reshape(NW, NCHUNK, CHUNK)
    e = _gather(emb, idx)
    e = e.reshape(B, L * D)
    bg = (b_ih + b_hh).reshape(4 * H, 1)
    wc = jnp.concatenate([W_ih, W_hh, bg], axis=1)
    out = _lstm_mlp(
        e, wc,
        W1, b1.reshape(-1, 1), W2, b2.reshape(-1, 1),
        W3, b3.reshape(-1, 1), W4, b4.reshape(-1, 1),
        W5, b5.reshape(1, 1))
    return out.reshape(B, 1)
